# Initial kernel scaffold; baseline (speedup 1.0000x reference)
import jax
import jax.numpy as jnp
from jax.experimental import pallas as pl


def kernel(x, edge_index, Wl0, bl0, Wr0, Wl1, bl1, Wr1, Wl2, bl2, Wr2):
    raise NotImplementedError("write your pallas kernel here")


# speedup vs baseline: 11.1744x; 11.1744x over previous
import functools

import jax
import jax.numpy as jnp
from jax import lax
from jax.experimental import pallas as pl
from jax.experimental.pallas import tpu as pltpu
from jax.experimental.pallas import tpu_sc as plsc

_NC = 2
_NS = 16
_B = 80


def _make_agg(N, D, E, with_cnt):
    NW = _NC * _NS
    n_rows = E // _B
    rpw = n_rows // NW
    gb = 25
    ngroup = rpw // gb
    npw = N // _NS
    mesh = plsc.VectorSubcoreMesh(core_axis_name="c", subcore_axis_name="s")

    out_type = [jax.ShapeDtypeStruct((N, D), jnp.float32),
                jax.ShapeDtypeStruct((N, D), jnp.float32)]
    scratch = [
        pltpu.VMEM((gb, _B), jnp.int32),
        pltpu.VMEM((gb, _B), jnp.int32),
        pltpu.VMEM((_B, D), jnp.float32),
        pltpu.VMEM((_B, D), jnp.float32),
        pltpu.VMEM_SHARED((N, D), jnp.float32),
        pltpu.SemaphoreType.DMA,
        pltpu.SemaphoreType.DMA,
    ]
    if with_cnt:
        out_type += [jax.ShapeDtypeStruct((N, 16), jnp.float32),
                     jax.ShapeDtypeStruct((N, 16), jnp.float32)]
        scratch += [pltpu.VMEM((_B, 16), jnp.float32),
                    pltpu.VMEM_SHARED((N, 16), jnp.float32)]

    def body(src_hbm, dst_hbm, h_hbm, z_nd, *rest):
        if with_cnt:
            (z16, ones_hbm, out0, out1, cnt_o0, cnt_o1,
             src_v, dst_v, buf0, buf1, acc, sem0, sem1, ones_v, cnt_sh) = rest
        else:
            (out0, out1, src_v, dst_v, buf0, buf1, acc, sem0, sem1) = rest
        cid = lax.axis_index("c")
        sid = lax.axis_index("s")
        wid = cid * _NS + sid

        pltpu.sync_copy(z_nd.at[pl.ds(sid * npw, npw)],
                        acc.at[pl.ds(sid * npw, npw)])
        if with_cnt:
            pltpu.sync_copy(z16.at[pl.ds(sid * npw, npw)],
                            cnt_sh.at[pl.ds(sid * npw, npw)])
            pltpu.sync_copy(ones_hbm, ones_v)
        plsc.subcore_barrier()

        def gather(j, buf, sem):
            pltpu.async_copy(h_hbm.at[src_v.at[j]], buf, sem)

        def wait(buf, sem):
            pltpu.make_async_copy(h_hbm.at[src_v.at[0]], buf, sem).wait()

        def scat(j, buf):
            pltpu.sync_copy(buf, acc.at[dst_v.at[j]], add=True)
            if with_cnt:
                pltpu.sync_copy(ones_v, cnt_sh.at[dst_v.at[j]], add=True)

        @pl.loop(0, ngroup)
        def _(g):
            base = wid * rpw + g * gb
            pltpu.sync_copy(src_hbm.at[pl.ds(base, gb)], src_v)
            pltpu.sync_copy(dst_hbm.at[pl.ds(base, gb)], dst_v)
            gather(0, buf0, sem0)

            @pl.loop(0, (gb - 1) // 2)
            def _(t):
                j = 2 * t
                gather(j + 1, buf1, sem1)
                wait(buf0, sem0)
                scat(j, buf0)
                gather(j + 2, buf0, sem0)
                wait(buf1, sem1)
                scat(j + 1, buf1)

            wait(buf0, sem0)
            scat(gb - 1, buf0)

        plsc.subcore_barrier()

        @pl.when(cid == 0)
        def _():
            pltpu.sync_copy(acc.at[pl.ds(sid * npw, npw)],
                            out0.at[pl.ds(sid * npw, npw)])
            if with_cnt:
                pltpu.sync_copy(cnt_sh.at[pl.ds(sid * npw, npw)],
                                cnt_o0.at[pl.ds(sid * npw, npw)])

        @pl.when(cid == 1)
        def _():
            pltpu.sync_copy(acc.at[pl.ds(sid * npw, npw)],
                            out1.at[pl.ds(sid * npw, npw)])
            if with_cnt:
                pltpu.sync_copy(cnt_sh.at[pl.ds(sid * npw, npw)],
                                cnt_o1.at[pl.ds(sid * npw, npw)])

    return pl.kernel(
        body, out_type=out_type, mesh=mesh, scratch_types=scratch,
        compiler_params=pltpu.CompilerParams(use_tc_tiling_on_sc=False))


def _mm_body(a0_ref, a1_ref, c0_ref, c1_ref, h_ref, wl_ref, wr_ref, bl_ref,
             pre_ref, act_ref):
    s = a0_ref[...] + a1_ref[...]
    cnt = c0_ref[...] + c1_ref[...]
    c = jnp.maximum(cnt[:, 0:1], 1.0)
    m = s / c
    pre = (jnp.dot(m, wl_ref[...], preferred_element_type=jnp.float32)
           + jnp.dot(h_ref[...], wr_ref[...], preferred_element_type=jnp.float32)
           + bl_ref[...])
    pre_ref[...] = pre
    act_ref[...] = jnp.maximum(pre, 0.0)


def _mm(a0, a1, c0, c1, h, Wl, Wr, bl):
    N, D = h.shape
    R = 1000
    return pl.pallas_call(
        _mm_body,
        grid=(N // R,),
        in_specs=[
            pl.BlockSpec((R, D), lambda i: (i, 0)),
            pl.BlockSpec((R, D), lambda i: (i, 0)),
            pl.BlockSpec((R, 16), lambda i: (i, 0)),
            pl.BlockSpec((R, 16), lambda i: (i, 0)),
            pl.BlockSpec((R, D), lambda i: (i, 0)),
            pl.BlockSpec((D, D), lambda i: (0, 0)),
            pl.BlockSpec((D, D), lambda i: (0, 0)),
            pl.BlockSpec((1, D), lambda i: (0, 0)),
        ],
        out_specs=[pl.BlockSpec((R, D), lambda i: (i, 0)),
                   pl.BlockSpec((R, D), lambda i: (i, 0))],
        out_shape=[jax.ShapeDtypeStruct((N, D), jnp.float32),
                   jax.ShapeDtypeStruct((N, D), jnp.float32)],
    )(a0, a1, c0, c1, h, Wl, Wr, bl.reshape(1, D))


def kernel(x, edge_index, Wl0, bl0, Wr0, Wl1, bl1, Wr1, Wl2, bl2, Wr2):
    N, D = x.shape
    E = edge_index.shape[1]
    src = edge_index[0].reshape(E // _B, _B)
    dst = edge_index[1].reshape(E // _B, _B)
    z_nd = jnp.zeros((N, D), jnp.float32)
    z16 = jnp.zeros((N, 16), jnp.float32)
    ones_b = jnp.ones((_B, 16), jnp.float32)

    agg_cnt = _make_agg(N, D, E, True)
    agg = _make_agg(N, D, E, False)

    a0, a1, c0, c1 = agg_cnt(src, dst, x, z_nd, z16, ones_b)
    pre0, h1 = _mm(a0, a1, c0, c1, x, Wl0, Wr0, bl0)
    a0, a1 = agg(src, dst, h1, z_nd)
    pre1, h2 = _mm(a0, a1, c0, c1, h1, Wl1, Wr1, bl1)
    a0, a1 = agg(src, dst, h2, z_nd)
    pre2, _ = _mm(a0, a1, c0, c1, h2, Wl2, Wr2, bl2)
    return (pre2, pre1)
